# Initial kernel scaffold; baseline (speedup 1.0000x reference)
import jax
import jax.numpy as jnp
from jax.experimental import pallas as pl


def kernel(x, edge_index, W_neigh, b_neigh, W_lin, b_lin):
    raise NotImplementedError("write your pallas kernel here")


# speedup vs baseline: 2.7382x; 2.7382x over previous
import functools

import jax
import jax.numpy as jnp
from jax import lax
from jax.experimental import pallas as pl
from jax.experimental.pallas import tpu as pltpu
from jax.experimental.pallas import tpu_sc as plsc

N_NODES = 10000
D = 256
DH = 128
DP = 144
E_PAD = 163840
CHUNK = 128
N_CHUNKS = E_PAD // 16 // CHUNK
ROWS_PER_TILE = N_NODES // 16
ZROWS = 125
TBL_ROWS = N_NODES + 8


def _sc_body(xlo, xhi, srcs, dsts, out, sbuf, dbuf, rows, zbuf, sem, acc):
    c = lax.axis_index("c")
    s = lax.axis_index("s")

    zeros16 = jnp.zeros((16,), jnp.float32)

    def zrow(r, _):
        for j in range(DP // 16):
            zbuf[r, pl.ds(j * 16, 16)] = zeros16
        return 0

    lax.fori_loop(0, ZROWS, zrow, 0)
    for k in range(ROWS_PER_TILE // ZROWS):
        pltpu.sync_copy(zbuf, acc.at[pl.ds(s * ROWS_PER_TILE + k * ZROWS, ZROWS)])
    plsc.subcore_barrier()

    ebase = s * (N_CHUNKS * CHUNK)

    def chunk(g, _):
        off = ebase + g * CHUNK
        pltpu.sync_copy(srcs.at[pl.ds(off, CHUNK)], sbuf)
        pltpu.sync_copy(dsts.at[pl.ds(off, CHUNK)], dbuf)

        @pl.when(c == 0)
        def _():
            pltpu.async_copy(xlo.at[sbuf], rows, sem).wait()

        @pl.when(c == 1)
        def _():
            pltpu.async_copy(xhi.at[sbuf], rows, sem).wait()

        pltpu.sync_copy(rows, acc.at[dbuf], add=True)
        return 0

    lax.fori_loop(0, N_CHUNKS, chunk, 0)
    plsc.subcore_barrier()

    rbase = s * ROWS_PER_TILE
    pltpu.sync_copy(acc.at[pl.ds(rbase, ROWS_PER_TILE)],
                    out.at[pl.ds(c * N_NODES + rbase, ROWS_PER_TILE)])


def _sc_aggregate(xlo, xhi, srcs, dsts):
    mesh = plsc.VectorSubcoreMesh(core_axis_name="c", subcore_axis_name="s")
    return pl.kernel(
        _sc_body,
        out_type=jax.ShapeDtypeStruct((2 * N_NODES, DP), jnp.float32),
        mesh=mesh,
        scratch_types=[
            pltpu.VMEM((CHUNK,), jnp.int32),
            pltpu.VMEM((CHUNK,), jnp.int32),
            pltpu.VMEM((CHUNK, DP), jnp.float32),
            pltpu.VMEM((ZROWS, DP), jnp.float32),
            pltpu.SemaphoreType.DMA,
            pltpu.VMEM_SHARED((N_NODES, DP), jnp.float32),
        ],
        compiler_params=pltpu.CompilerParams(use_tc_tiling_on_sc=False),
    )(xlo, xhi, srcs, dsts)


def _tc_body(sum_ref, deg_ref, x_ref, wn_ref, bn_ref, wl_ref, bl_ref, o_ref):
    recip = 1.0 / jnp.maximum(deg_ref[...], 1.0)
    h_in = sum_ref[...] * recip
    h = lax.dot_general(h_in, wn_ref[...], (((1,), (1,)), ((), ())),
                        preferred_element_type=jnp.float32) + bn_ref[...]
    l = lax.dot_general(x_ref[...], wl_ref[...], (((1,), (1,)), ((), ())),
                        preferred_element_type=jnp.float32) + bl_ref[...]
    h = jnp.where(h > 0, h, jnp.exp(jnp.minimum(h, 0.0)) - 1.0)
    l = jnp.where(l > 0, l, jnp.exp(jnp.minimum(l, 0.0)) - 1.0)
    o_ref[...] = h + l


def _tc_finish(summed, deg, x, w_neigh, b_neigh, w_lin, b_lin):
    blk = 1000
    grid = N_NODES // blk
    return pl.pallas_call(
        _tc_body,
        grid=(grid,),
        in_specs=[
            pl.BlockSpec((blk, D), lambda i: (i, 0)),
            pl.BlockSpec((blk, 1), lambda i: (i, 0)),
            pl.BlockSpec((blk, D), lambda i: (i, 0)),
            pl.BlockSpec((D, D), lambda i: (0, 0)),
            pl.BlockSpec((1, D), lambda i: (0, 0)),
            pl.BlockSpec((D, D), lambda i: (0, 0)),
            pl.BlockSpec((1, D), lambda i: (0, 0)),
        ],
        out_specs=pl.BlockSpec((blk, D), lambda i: (i, 0)),
        out_shape=jax.ShapeDtypeStruct((N_NODES, D), jnp.float32),
    )(summed, deg, x, w_neigh, b_neigh.reshape(1, D), w_lin, b_lin.reshape(1, D))


@jax.jit
def kernel(x, edge_index, W_neigh, b_neigh, W_lin, b_lin):
    src = edge_index[0].astype(jnp.int32)
    dst = edge_index[1].astype(jnp.int32)
    e = src.shape[0]
    src_p = jnp.concatenate(
        [src, jnp.full((E_PAD - e,), N_NODES, jnp.int32)])
    dst_p = jnp.concatenate([dst, jnp.zeros((E_PAD - e,), jnp.int32)])

    ones = jnp.ones((N_NODES, 1), jnp.float32)
    zpad = jnp.zeros((N_NODES, DP - DH - 1), jnp.float32)
    zrows = jnp.zeros((TBL_ROWS - N_NODES, DP), jnp.float32)
    xlo = jnp.concatenate([jnp.concatenate([x[:, :DH], ones, zpad], 1), zrows], 0)
    xhi = jnp.concatenate([jnp.concatenate([x[:, DH:], ones, zpad], 1), zrows], 0)

    agg = _sc_aggregate(xlo, xhi, src_p, dst_p)
    summed = jnp.concatenate([agg[:N_NODES, :DH], agg[N_NODES:, :DH]], 1)
    deg = agg[:N_NODES, DH:DH + 1]
    return _tc_finish(summed, deg, x, W_neigh, b_neigh, W_lin, b_lin)


# speedup vs baseline: 3.4425x; 1.2572x over previous
import functools

import jax
import jax.numpy as jnp
from jax import lax
from jax.experimental import pallas as pl
from jax.experimental.pallas import tpu as pltpu
from jax.experimental.pallas import tpu_sc as plsc

N_NODES = 10000
D = 256
DH = 128
DP = 144
E_PAD = 163840
CHUNK = 128
N_CHUNKS = E_PAD // 16 // CHUNK
ROWS_PER_TILE = N_NODES // 16
ZROWS = 125
TBL_ROWS = N_NODES + 8


NBUF = 2
EPT = E_PAD // 16


def _sc_body(xlo, xhi, srcs, dsts, out, sbuf0, sbuf1, dbuf0, dbuf1,
             rows0, rows1, sem0, sem1, acc):
    c = lax.axis_index("c")
    s = lax.axis_index("s")
    sbufs = (sbuf0, sbuf1)
    dbufs = (dbuf0, dbuf1)
    rowss = (rows0, rows1)
    sems = (sem0, sem1)

    ebase = s * EPT

    zeros16 = jnp.zeros((16,), jnp.float32)

    def zrow(r, _):
        for j in range(DP // 16):
            rows0[r, pl.ds(j * 16, 16)] = zeros16
        return 0

    lax.fori_loop(0, ZROWS, zrow, 0)
    for k in range(ROWS_PER_TILE // ZROWS):
        pltpu.sync_copy(rows0.at[pl.ds(0, ZROWS)],
                        acc.at[pl.ds(s * ROWS_PER_TILE + k * ZROWS, ZROWS)])
    plsc.subcore_barrier()

    def fire(g, b):
        off = ebase + g * CHUNK
        pltpu.sync_copy(srcs.at[pl.ds(off, CHUNK)], sbufs[b])
        pltpu.sync_copy(dsts.at[pl.ds(off, CHUNK)], dbufs[b])

        @pl.when(c == 0)
        def _():
            pltpu.async_copy(xlo.at[sbufs[b]], rowss[b], sems[b])

        @pl.when(c == 1)
        def _():
            pltpu.async_copy(xhi.at[sbufs[b]], rowss[b], sems[b])

    def drain(b):
        pltpu.make_async_copy(xlo.at[sbufs[b]], rowss[b], sems[b]).wait()

    for b in range(NBUF):
        fire(b, b)

    n_steps = N_CHUNKS // NBUF

    def step(t, _):
        for b in range(NBUF):
            drain(b)
            pltpu.sync_copy(rowss[b], acc.at[dbufs[b]], add=True)

            @pl.when(t < n_steps - 1)
            def _():
                fire(t * NBUF + b + NBUF, b)
        return 0

    lax.fori_loop(0, n_steps, step, 0)
    plsc.subcore_barrier()

    rbase = s * ROWS_PER_TILE
    pltpu.sync_copy(acc.at[pl.ds(rbase, ROWS_PER_TILE)],
                    out.at[pl.ds(c * N_NODES + rbase, ROWS_PER_TILE)])


def _sc_aggregate(xlo, xhi, srcs, dsts):
    mesh = plsc.VectorSubcoreMesh(core_axis_name="c", subcore_axis_name="s")
    return pl.kernel(
        _sc_body,
        out_type=jax.ShapeDtypeStruct((2 * N_NODES, DP), jnp.float32),
        mesh=mesh,
        scratch_types=[
            pltpu.VMEM((CHUNK,), jnp.int32),
            pltpu.VMEM((CHUNK,), jnp.int32),
            pltpu.VMEM((CHUNK,), jnp.int32),
            pltpu.VMEM((CHUNK,), jnp.int32),
            pltpu.VMEM((CHUNK, DP), jnp.float32),
            pltpu.VMEM((CHUNK, DP), jnp.float32),
            pltpu.SemaphoreType.DMA,
            pltpu.SemaphoreType.DMA,
            pltpu.VMEM_SHARED((N_NODES, DP), jnp.float32),
        ],
        compiler_params=pltpu.CompilerParams(use_tc_tiling_on_sc=False),
    )(xlo, xhi, srcs, dsts)


def _tc_body(sum_ref, deg_ref, x_ref, wn_ref, bn_ref, wl_ref, bl_ref, o_ref):
    recip = 1.0 / jnp.maximum(deg_ref[...], 1.0)
    h_in = sum_ref[...] * recip
    h = lax.dot_general(h_in, wn_ref[...], (((1,), (1,)), ((), ())),
                        preferred_element_type=jnp.float32) + bn_ref[...]
    l = lax.dot_general(x_ref[...], wl_ref[...], (((1,), (1,)), ((), ())),
                        preferred_element_type=jnp.float32) + bl_ref[...]
    h = jnp.where(h > 0, h, jnp.exp(jnp.minimum(h, 0.0)) - 1.0)
    l = jnp.where(l > 0, l, jnp.exp(jnp.minimum(l, 0.0)) - 1.0)
    o_ref[...] = h + l


def _tc_finish(summed, deg, x, w_neigh, b_neigh, w_lin, b_lin):
    blk = 1000
    grid = N_NODES // blk
    return pl.pallas_call(
        _tc_body,
        grid=(grid,),
        in_specs=[
            pl.BlockSpec((blk, D), lambda i: (i, 0)),
            pl.BlockSpec((blk, 1), lambda i: (i, 0)),
            pl.BlockSpec((blk, D), lambda i: (i, 0)),
            pl.BlockSpec((D, D), lambda i: (0, 0)),
            pl.BlockSpec((1, D), lambda i: (0, 0)),
            pl.BlockSpec((D, D), lambda i: (0, 0)),
            pl.BlockSpec((1, D), lambda i: (0, 0)),
        ],
        out_specs=pl.BlockSpec((blk, D), lambda i: (i, 0)),
        out_shape=jax.ShapeDtypeStruct((N_NODES, D), jnp.float32),
    )(summed, deg, x, w_neigh, b_neigh.reshape(1, D), w_lin, b_lin.reshape(1, D))


@jax.jit
def kernel(x, edge_index, W_neigh, b_neigh, W_lin, b_lin):
    src = edge_index[0].astype(jnp.int32)
    dst = edge_index[1].astype(jnp.int32)
    e = src.shape[0]
    src_p = jnp.concatenate(
        [src, jnp.full((E_PAD - e,), N_NODES, jnp.int32)])
    dst_p = jnp.concatenate([dst, jnp.zeros((E_PAD - e,), jnp.int32)])

    ones = jnp.ones((N_NODES, 1), jnp.float32)
    zpad = jnp.zeros((N_NODES, DP - DH - 1), jnp.float32)
    zrows = jnp.zeros((TBL_ROWS - N_NODES, DP), jnp.float32)
    xlo = jnp.concatenate([jnp.concatenate([x[:, :DH], ones, zpad], 1), zrows], 0)
    xhi = jnp.concatenate([jnp.concatenate([x[:, DH:], ones, zpad], 1), zrows], 0)

    agg = _sc_aggregate(xlo, xhi, src_p, dst_p)
    summed = jnp.concatenate([agg[:N_NODES, :DH], agg[N_NODES:, :DH]], 1)
    deg = agg[:N_NODES, DH:DH + 1]
    return _tc_finish(summed, deg, x, W_neigh, b_neigh, W_lin, b_lin)


# speedup vs baseline: 3.6606x; 1.0633x over previous
import functools

import jax
import jax.numpy as jnp
from jax import lax
from jax.experimental import pallas as pl
from jax.experimental.pallas import tpu as pltpu
from jax.experimental.pallas import tpu_sc as plsc

N_NODES = 10000
D = 256
DH = 128
DP = 144
E_PAD = 163840
CHUNK = 128
N_CHUNKS = E_PAD // 16 // CHUNK
ROWS_PER_TILE = N_NODES // 16
ZROWS = 125
TBL_ROWS = N_NODES + 8


NBUF = 2
EPT = E_PAD // 16
SUP = 8
SUPE = SUP * CHUNK
N_SUP = EPT // SUPE


def _sc_body(xlo, xhi, srcs, dsts, out, sbig0, sbig1, dbig0, dbig1,
             rows0, rows1, sem0, sem1, isem0, isem1, acc):
    c = lax.axis_index("c")
    s = lax.axis_index("s")
    sbigs = (sbig0, sbig1)
    dbigs = (dbig0, dbig1)
    rowss = (rows0, rows1)
    sems = (sem0, sem1)
    isems = (isem0, isem1)

    ebase = s * EPT
    cbase = ebase // CHUNK

    zeros16 = jnp.zeros((16,), jnp.float32)

    def zrow(r, _):
        for j in range(DP // 16):
            rows0[r, pl.ds(j * 16, 16)] = zeros16
        return 0

    lax.fori_loop(0, ZROWS, zrow, 0)
    for k in range(ROWS_PER_TILE // ZROWS):
        pltpu.sync_copy(rows0.at[pl.ds(0, ZROWS)],
                        acc.at[pl.ds(s * ROWS_PER_TILE + k * ZROWS, ZROWS)])

    def fire_idx(sup, ib):
        pltpu.async_copy(srcs.at[pl.ds(ebase + sup * SUPE, SUPE)],
                         sbigs[ib], isems[ib])
        pltpu.async_copy(dsts.at[pl.ds(cbase + sup * SUP, SUP)],
                         dbigs[ib], isems[ib])

    def wait_idx(ib):
        pltpu.make_async_copy(srcs.at[pl.ds(0, SUPE)], sbigs[ib],
                              isems[ib]).wait()
        pltpu.make_async_copy(dsts.at[pl.ds(0, SUP)], dbigs[ib],
                              isems[ib]).wait()

    def fire(ib, k, b):
        idx = sbigs[ib].at[pl.ds(k * CHUNK, CHUNK)]

        @pl.when(c == 0)
        def _():
            pltpu.async_copy(xlo.at[idx], rowss[b], sems[b])

        @pl.when(c == 1)
        def _():
            pltpu.async_copy(xhi.at[idx], rowss[b], sems[b])

    def drain(b):
        pltpu.make_async_copy(xlo.at[sbigs[0].at[pl.ds(0, CHUNK)]],
                              rowss[b], sems[b]).wait()

    fire_idx(0, 0)
    plsc.subcore_barrier()

    def sup_body(h, ib):
        sup = 2 * h + ib
        wait_idx(ib)

        @pl.when(sup < N_SUP - 1)
        def _():
            fire_idx(sup + 1, 1 - ib)

        fire(ib, 0, 0)
        fire(ib, 1, 1)
        for k in range(SUP):
            b = k % NBUF
            drain(b)
            pltpu.sync_copy(rowss[b], acc.at[dbigs[ib].at[k]], add=True)
            if k + NBUF < SUP:
                fire(ib, k + NBUF, b)

    def step(h, _):
        for ib in range(2):
            sup_body(h, ib)
        return 0

    lax.fori_loop(0, N_SUP // 2, step, 0)
    plsc.subcore_barrier()

    rbase = s * ROWS_PER_TILE
    pltpu.sync_copy(acc.at[pl.ds(rbase, ROWS_PER_TILE)],
                    out.at[pl.ds(c * N_NODES + rbase, ROWS_PER_TILE)])


def _sc_aggregate(xlo, xhi, srcs, dsts):
    mesh = plsc.VectorSubcoreMesh(core_axis_name="c", subcore_axis_name="s")
    return pl.kernel(
        _sc_body,
        out_type=jax.ShapeDtypeStruct((2 * N_NODES, DP), jnp.float32),
        mesh=mesh,
        scratch_types=[
            pltpu.VMEM((SUPE,), jnp.int32),
            pltpu.VMEM((SUPE,), jnp.int32),
            pltpu.VMEM((SUP, CHUNK), jnp.int32),
            pltpu.VMEM((SUP, CHUNK), jnp.int32),
            pltpu.VMEM((CHUNK, DP), jnp.float32),
            pltpu.VMEM((CHUNK, DP), jnp.float32),
            pltpu.SemaphoreType.DMA,
            pltpu.SemaphoreType.DMA,
            pltpu.SemaphoreType.DMA,
            pltpu.SemaphoreType.DMA,
            pltpu.VMEM_SHARED((N_NODES, DP), jnp.float32),
        ],
        compiler_params=pltpu.CompilerParams(use_tc_tiling_on_sc=False),
    )(xlo, xhi, srcs, dsts)


def _tc_body(sum_ref, deg_ref, x_ref, wn_ref, bn_ref, wl_ref, bl_ref, o_ref):
    recip = 1.0 / jnp.maximum(deg_ref[...], 1.0)
    h_in = sum_ref[...] * recip
    h = lax.dot_general(h_in, wn_ref[...], (((1,), (1,)), ((), ())),
                        preferred_element_type=jnp.float32) + bn_ref[...]
    l = lax.dot_general(x_ref[...], wl_ref[...], (((1,), (1,)), ((), ())),
                        preferred_element_type=jnp.float32) + bl_ref[...]
    h = jnp.where(h > 0, h, jnp.exp(jnp.minimum(h, 0.0)) - 1.0)
    l = jnp.where(l > 0, l, jnp.exp(jnp.minimum(l, 0.0)) - 1.0)
    o_ref[...] = h + l


def _tc_finish(summed, deg, x, w_neigh, b_neigh, w_lin, b_lin):
    blk = 1000
    grid = N_NODES // blk
    return pl.pallas_call(
        _tc_body,
        grid=(grid,),
        in_specs=[
            pl.BlockSpec((blk, D), lambda i: (i, 0)),
            pl.BlockSpec((blk, 1), lambda i: (i, 0)),
            pl.BlockSpec((blk, D), lambda i: (i, 0)),
            pl.BlockSpec((D, D), lambda i: (0, 0)),
            pl.BlockSpec((1, D), lambda i: (0, 0)),
            pl.BlockSpec((D, D), lambda i: (0, 0)),
            pl.BlockSpec((1, D), lambda i: (0, 0)),
        ],
        out_specs=pl.BlockSpec((blk, D), lambda i: (i, 0)),
        out_shape=jax.ShapeDtypeStruct((N_NODES, D), jnp.float32),
    )(summed, deg, x, w_neigh, b_neigh.reshape(1, D), w_lin, b_lin.reshape(1, D))


@jax.jit
def kernel(x, edge_index, W_neigh, b_neigh, W_lin, b_lin):
    src = edge_index[0].astype(jnp.int32)
    dst = edge_index[1].astype(jnp.int32)
    e = src.shape[0]
    src_p = jnp.concatenate(
        [src, jnp.full((E_PAD - e,), N_NODES, jnp.int32)])
    dst_p = jnp.concatenate(
        [dst, jnp.zeros((E_PAD - e,), jnp.int32)]).reshape(E_PAD // CHUNK, CHUNK)

    ones = jnp.ones((N_NODES, 1), jnp.float32)
    zpad = jnp.zeros((N_NODES, DP - DH - 1), jnp.float32)
    zrows = jnp.zeros((TBL_ROWS - N_NODES, DP), jnp.float32)
    xlo = jnp.concatenate([jnp.concatenate([x[:, :DH], ones, zpad], 1), zrows], 0)
    xhi = jnp.concatenate([jnp.concatenate([x[:, DH:], ones, zpad], 1), zrows], 0)

    agg = _sc_aggregate(xlo, xhi, src_p, dst_p)
    summed = jnp.concatenate([agg[:N_NODES, :DH], agg[N_NODES:, :DH]], 1)
    deg = agg[:N_NODES, DH:DH + 1]
    return _tc_finish(summed, deg, x, W_neigh, b_neigh, W_lin, b_lin)
